# baseline (speedup 1.0000x reference)
import functools

import jax
import jax.numpy as jnp
from jax import lax
from jax.experimental import pallas as pl
from jax.experimental.pallas import tpu as pltpu
from jax.experimental.pallas import tpu_sc as plsc

_HIDDEN = 768
_B, _L = 4096, 20
_LP = 32
_LG = 24

_NC, _NS, _LANES = 2, 16, 16
_NW = _NC * _NS
_ROWS_W = _B // _NW

_mesh = plsc.VectorSubcoreMesh(core_axis_name="c", subcore_axis_name="s")


@functools.partial(
    pl.kernel,
    mesh=_mesh,
    out_type=jax.ShapeDtypeStruct((_B, _LG, _HIDDEN), jnp.float32),
    scratch_types=[
        pltpu.VMEM((_ROWS_W * _LP,), jnp.int32),
        pltpu.VMEM((_ROWS_W * _LP,), jnp.int32),
        pltpu.VMEM((_LG, _HIDDEN), jnp.float32),
        pltpu.VMEM((_LG, _HIDDEN), jnp.float32),
        pltpu.VMEM((_LG, _HIDDEN), jnp.float32),
        pltpu.VMEM((_LG, _HIDDEN), jnp.float32),
        pltpu.SemaphoreType.DMA,
        pltpu.SemaphoreType.DMA,
        pltpu.SemaphoreType.DMA,
        pltpu.SemaphoreType.DMA,
        pltpu.SemaphoreType.DMA,
        pltpu.SemaphoreType.DMA,
    ],
)
def _embed_sum(gsn_ids_hbm, route_ids_hbm, gsn_hbm, route_hbm, out_hbm,
               gidx, ridx, gbuf0, rbuf0, gbuf1, rbuf1,
               sem_g0, sem_g1, sem_r0, sem_r1, sem_o0, sem_o1):
    wid = lax.axis_index("s") * _NC + lax.axis_index("c")
    row_base = wid * _ROWS_W

    pltpu.sync_copy(gsn_ids_hbm.at[wid], gidx)
    pltpu.sync_copy(route_ids_hbm.at[wid], ridx)

    def add_rows(gbuf, rbuf):
        def row_body(i, c):
            for j in range(_HIDDEN // _LANES):
                sl = pl.ds(j * _LANES, _LANES)
                gbuf[i, sl] = gbuf[i, sl] + rbuf[i, sl]
            return c
        lax.fori_loop(0, _LG, row_body, 0)

    def start_gathers(b, gbuf, rbuf, sg, sr):
        isl = pl.ds(b * _LP, _LG)
        dg = pltpu.async_copy(gsn_hbm.at[gidx.at[isl]], gbuf, sg)
        dr = pltpu.async_copy(route_hbm.at[ridx.at[isl]], rbuf, sr)
        return dg, dr

    def start_writeback(b, gbuf, so):
        return pltpu.async_copy(gbuf, out_hbm.at[row_base + b], so)

    def group_body(g, carry):
        b0 = 2 * g
        b1 = b0 + 1
        dg0, dr0 = start_gathers(b0, gbuf0, rbuf0, sem_g0, sem_r0)
        dg1, dr1 = start_gathers(b1, gbuf1, rbuf1, sem_g1, sem_r1)

        dg0.wait()
        dr0.wait()
        add_rows(gbuf0, rbuf0)
        wb0 = start_writeback(b0, gbuf0, sem_o0)

        dg1.wait()
        dr1.wait()
        add_rows(gbuf1, rbuf1)
        wb1 = start_writeback(b1, gbuf1, sem_o1)

        wb0.wait()
        wb1.wait()
        return carry

    lax.fori_loop(0, _ROWS_W // 2, group_body, 0)


def _pad_ids(ids):
    ids = ids.astype(jnp.int32)
    ids = jnp.pad(ids, ((0, 0), (0, _LP - _L)))
    return ids.reshape(_NW, _ROWS_W * _LP)


def kernel(gsn_ids, route_ids, gsn_table, route_table):
    out = _embed_sum(_pad_ids(gsn_ids), _pad_ids(route_ids),
                     gsn_table, route_table)
    return out[:, :_L, :]


# speedup vs baseline: 1.0015x; 1.0015x over previous
import functools

import jax
import jax.numpy as jnp
from jax import lax
from jax.experimental import pallas as pl
from jax.experimental.pallas import tpu as pltpu
from jax.experimental.pallas import tpu_sc as plsc

_HIDDEN = 768
_B, _L = 4096, 20
_LP = 32
_LG = 24

_NC, _NS, _LANES = 2, 16, 16
_NW = _NC * _NS
_ROWS_W = _B // _NW

_mesh = plsc.VectorSubcoreMesh(core_axis_name="c", subcore_axis_name="s")


@functools.partial(
    pl.kernel,
    mesh=_mesh,
    out_type=jax.ShapeDtypeStruct((_B * _LG, _HIDDEN), jnp.float32),
    scratch_types=[
        pltpu.VMEM((_ROWS_W * _LP,), jnp.int32),
        pltpu.VMEM((_ROWS_W * _LP,), jnp.int32),
        pltpu.VMEM((_LG, _HIDDEN), jnp.float32),
        pltpu.VMEM((_LG, _HIDDEN), jnp.float32),
        pltpu.VMEM((_LG, _HIDDEN), jnp.float32),
        pltpu.VMEM((_LG, _HIDDEN), jnp.float32),
        pltpu.SemaphoreType.DMA,
        pltpu.SemaphoreType.DMA,
        pltpu.SemaphoreType.DMA,
        pltpu.SemaphoreType.DMA,
        pltpu.SemaphoreType.DMA,
        pltpu.SemaphoreType.DMA,
    ],
)
def _embed_sum(gsn_ids_hbm, route_ids_hbm, gsn_hbm, route_hbm, out_hbm,
               gidx, ridx, gbuf0, rbuf0, gbuf1, rbuf1,
               sem_g0, sem_g1, sem_r0, sem_r1, sem_o0, sem_o1):
    wid = lax.axis_index("s") * _NC + lax.axis_index("c")
    row_base = wid * _ROWS_W

    pltpu.sync_copy(gsn_ids_hbm.at[wid], gidx)
    pltpu.sync_copy(route_ids_hbm.at[wid], ridx)

    def add_rows(gbuf, rbuf):
        def row_body(i, c):
            for j in range(_HIDDEN // _LANES):
                sl = pl.ds(j * _LANES, _LANES)
                gbuf[i, sl] = gbuf[i, sl] + rbuf[i, sl]
            return c
        lax.fori_loop(0, _LG, row_body, 0)

    def start_gathers(b, gbuf, rbuf, sg, sr):
        isl = pl.ds(b * _LP, _LG)
        dg = pltpu.async_copy(gsn_hbm.at[gidx.at[isl]], gbuf, sg)
        dr = pltpu.async_copy(route_hbm.at[ridx.at[isl]], rbuf, sr)
        return dg, dr

    def start_writeback(b, gbuf, so):
        off = (row_base + b) * _LG
        return pltpu.async_copy(gbuf, out_hbm.at[pl.ds(off, _LG)], so)

    def group_body(g, carry):
        b0 = 2 * g
        b1 = b0 + 1
        dg0, dr0 = start_gathers(b0, gbuf0, rbuf0, sem_g0, sem_r0)
        dg1, dr1 = start_gathers(b1, gbuf1, rbuf1, sem_g1, sem_r1)

        dg0.wait()
        dr0.wait()
        add_rows(gbuf0, rbuf0)
        wb0 = start_writeback(b0, gbuf0, sem_o0)

        dg1.wait()
        dr1.wait()
        add_rows(gbuf1, rbuf1)
        wb1 = start_writeback(b1, gbuf1, sem_o1)

        wb0.wait()
        wb1.wait()
        return carry

    lax.fori_loop(0, _ROWS_W // 2, group_body, 0)


def _pad_ids(ids):
    ids = ids.astype(jnp.int32)
    ids = jnp.pad(ids, ((0, 0), (0, _LP - _L)))
    return ids.reshape(_NW, _ROWS_W * _LP)


def kernel(gsn_ids, route_ids, gsn_table, route_table):
    out = _embed_sum(_pad_ids(gsn_ids), _pad_ids(route_ids),
                     gsn_table, route_table)
    return out.reshape(_B, _LG, _HIDDEN)[:, :_L, :]


# speedup vs baseline: 1.7629x; 1.7602x over previous
import functools

import jax
import jax.numpy as jnp
from jax import lax
from jax.experimental import pallas as pl
from jax.experimental.pallas import tpu as pltpu
from jax.experimental.pallas import tpu_sc as plsc

_HIDDEN = 768
_B, _L = 4096, 20
_LP = 32
_LG = 24

_NC, _NS, _LANES = 2, 16, 16
_NW = _NC * _NS
_ROWS_W = _B // _NW

_mesh = plsc.VectorSubcoreMesh(core_axis_name="c", subcore_axis_name="s")


@functools.partial(
    pl.kernel,
    mesh=_mesh,
    out_type=jax.ShapeDtypeStruct((_B * _LG, _HIDDEN), jnp.float32),
    scratch_types=[
        pltpu.VMEM((_ROWS_W * _LP,), jnp.int32),
        pltpu.VMEM((_ROWS_W * _LP,), jnp.int32),
        pltpu.VMEM((_LG, _HIDDEN), jnp.float32),
        pltpu.VMEM((_LG, _HIDDEN), jnp.float32),
        pltpu.VMEM((_LG, _HIDDEN), jnp.float32),
        pltpu.VMEM((_LG, _HIDDEN), jnp.float32),
        pltpu.SemaphoreType.DMA,
        pltpu.SemaphoreType.DMA,
        pltpu.SemaphoreType.DMA,
        pltpu.SemaphoreType.DMA,
        pltpu.SemaphoreType.DMA,
        pltpu.SemaphoreType.DMA,
    ],
)
def _embed_sum(gsn_ids_hbm, route_ids_hbm, gsn_hbm, route_hbm, out_hbm,
               gidx, ridx, gbuf0, rbuf0, gbuf1, rbuf1,
               sem_g0, sem_g1, sem_r0, sem_r1, sem_o0, sem_o1):
    wid = lax.axis_index("s") * _NC + lax.axis_index("c")
    row_base = wid * _ROWS_W

    pltpu.sync_copy(gsn_ids_hbm.at[wid], gidx)
    pltpu.sync_copy(route_ids_hbm.at[wid], ridx)

    def add_rows(gbuf, rbuf):
        def row_body(i, c):
            for j in range(_HIDDEN // _LANES):
                sl = pl.ds(j * _LANES, _LANES)
                gbuf[i, sl] = gbuf[i, sl] + rbuf[i, sl]
            return c
        lax.fori_loop(0, _LG, row_body, 0)

    def start_gathers(b, gbuf, rbuf, sg, sr):
        isl = pl.ds(b * _LP, _LG)
        dg = pltpu.async_copy(gsn_hbm.at[gidx.at[isl]], gbuf, sg)
        dr = pltpu.async_copy(route_hbm.at[ridx.at[isl]], rbuf, sr)
        return dg, dr

    def start_writeback(b, gbuf, so):
        off = (row_base + b) * _LG
        return pltpu.async_copy(gbuf, out_hbm.at[pl.ds(off, _LG)], so)

    def group_body(g, carry):
        b0 = 2 * g
        b1 = b0 + 1
        dg0, dr0 = start_gathers(b0, gbuf0, rbuf0, sem_g0, sem_r0)
        dg1, dr1 = start_gathers(b1, gbuf1, rbuf1, sem_g1, sem_r1)

        dg0.wait()
        dr0.wait()
        add_rows(gbuf0, rbuf0)
        wb0 = start_writeback(b0, gbuf0, sem_o0)

        dg1.wait()
        dr1.wait()
        add_rows(gbuf1, rbuf1)
        wb1 = start_writeback(b1, gbuf1, sem_o1)

        wb0.wait()
        wb1.wait()
        return carry

    lax.fori_loop(0, _ROWS_W // 2, group_body, 0)


def _pad_ids(ids):
    ids = ids.astype(jnp.int32)
    ids = jnp.pad(ids, ((0, 0), (0, _LP - _L)), mode='edge')
    return ids.reshape(_NW, _ROWS_W * _LP)


def kernel(gsn_ids, route_ids, gsn_table, route_table):
    out = _embed_sum(_pad_ids(gsn_ids), _pad_ids(route_ids),
                     gsn_table, route_table)
    return out.reshape(_B, _LG, _HIDDEN)[:, :_L, :]


# speedup vs baseline: 1.8928x; 1.0737x over previous
import functools

import jax
import jax.numpy as jnp
from jax import lax
from jax.experimental import pallas as pl
from jax.experimental.pallas import tpu as pltpu
from jax.experimental.pallas import tpu_sc as plsc

_HIDDEN = 768
_B, _L = 4096, 20
_LP = 32
_LG = 24

_NC, _NS, _LANES = 2, 16, 16
_NW = _NC * _NS
_ROWS_W = _B // _NW

_mesh = plsc.VectorSubcoreMesh(core_axis_name="c", subcore_axis_name="s")


@functools.partial(
    pl.kernel,
    mesh=_mesh,
    out_type=jax.ShapeDtypeStruct((_B, _L, _HIDDEN), jnp.float32),
    scratch_types=[
        pltpu.VMEM((_ROWS_W * _LP,), jnp.int32),
        pltpu.VMEM((_ROWS_W * _LP,), jnp.int32),
        pltpu.VMEM((_LG, _HIDDEN), jnp.float32),
        pltpu.VMEM((_LG, _HIDDEN), jnp.float32),
        pltpu.VMEM((_L, _HIDDEN), jnp.float32),
        pltpu.VMEM((_LG, _HIDDEN), jnp.float32),
        pltpu.VMEM((_LG, _HIDDEN), jnp.float32),
        pltpu.VMEM((_L, _HIDDEN), jnp.float32),
        pltpu.SemaphoreType.DMA,
        pltpu.SemaphoreType.DMA,
        pltpu.SemaphoreType.DMA,
        pltpu.SemaphoreType.DMA,
        pltpu.SemaphoreType.DMA,
        pltpu.SemaphoreType.DMA,
    ],
)
def _embed_sum(gsn_ids_hbm, route_ids_hbm, gsn_hbm, route_hbm, out_hbm,
               gidx, ridx, gbuf0, rbuf0, obuf0, gbuf1, rbuf1, obuf1,
               sem_g0, sem_g1, sem_r0, sem_r1, sem_o0, sem_o1):
    wid = lax.axis_index("s") * _NC + lax.axis_index("c")
    row_base = wid * _ROWS_W

    pltpu.sync_copy(gsn_ids_hbm.at[wid], gidx)
    pltpu.sync_copy(route_ids_hbm.at[wid], ridx)

    def add_rows(gbuf, rbuf, obuf):
        def row_body(i, c):
            for j in range(_HIDDEN // _LANES):
                sl = pl.ds(j * _LANES, _LANES)
                obuf[i, sl] = gbuf[i, sl] + rbuf[i, sl]
            return c
        lax.fori_loop(0, _L, row_body, 0)

    def start_gathers(b, gbuf, rbuf, sg, sr):
        isl = pl.ds(b * _LP, _LG)
        dg = pltpu.async_copy(gsn_hbm.at[gidx.at[isl]], gbuf, sg)
        dr = pltpu.async_copy(route_hbm.at[ridx.at[isl]], rbuf, sr)
        return dg, dr

    def start_writeback(b, obuf, so):
        return pltpu.async_copy(obuf, out_hbm.at[row_base + b], so)

    def group_body(g, carry):
        b0 = 2 * g
        b1 = b0 + 1
        dg0, dr0 = start_gathers(b0, gbuf0, rbuf0, sem_g0, sem_r0)
        dg1, dr1 = start_gathers(b1, gbuf1, rbuf1, sem_g1, sem_r1)

        dg0.wait()
        dr0.wait()
        add_rows(gbuf0, rbuf0, obuf0)
        wb0 = start_writeback(b0, obuf0, sem_o0)

        dg1.wait()
        dr1.wait()
        add_rows(gbuf1, rbuf1, obuf1)
        wb1 = start_writeback(b1, obuf1, sem_o1)

        wb0.wait()
        wb1.wait()
        return carry

    lax.fori_loop(0, _ROWS_W // 2, group_body, 0)


def _pad_ids(ids):
    ids = ids.astype(jnp.int32)
    ids = jnp.pad(ids, ((0, 0), (0, _LP - _L)), mode='edge')
    return ids.reshape(_NW, _ROWS_W * _LP)


def kernel(gsn_ids, route_ids, gsn_table, route_table):
    return _embed_sum(_pad_ids(gsn_ids), _pad_ids(route_ids),
                      gsn_table, route_table)


# speedup vs baseline: 2.1686x; 1.1457x over previous
import functools

import jax
import jax.numpy as jnp
from jax import lax
from jax.experimental import pallas as pl
from jax.experimental.pallas import tpu as pltpu
from jax.experimental.pallas import tpu_sc as plsc

_HIDDEN = 768
_B, _L = 4096, 20
_LP = 32
_LG = 24

_NC, _NS, _LANES = 2, 16, 16
_NW = _NC * _NS
_ROWS_W = _B // _NW

_mesh = plsc.VectorSubcoreMesh(core_axis_name="c", subcore_axis_name="s")


@functools.partial(
    pl.kernel,
    mesh=_mesh,
    out_type=jax.ShapeDtypeStruct((_B, _L, _HIDDEN), jnp.float32),
    scratch_types=[
        pltpu.VMEM((_ROWS_W * _LP,), jnp.int32),
        pltpu.VMEM((_ROWS_W * _LP,), jnp.int32),
        pltpu.VMEM((_LG, _HIDDEN), jnp.float32),
        pltpu.VMEM((_LG, _HIDDEN), jnp.float32),
        pltpu.VMEM((_L, _HIDDEN), jnp.float32),
        pltpu.VMEM((_LG, _HIDDEN), jnp.float32),
        pltpu.VMEM((_LG, _HIDDEN), jnp.float32),
        pltpu.VMEM((_L, _HIDDEN), jnp.float32),
        pltpu.SemaphoreType.DMA,
        pltpu.SemaphoreType.DMA,
        pltpu.SemaphoreType.DMA,
        pltpu.SemaphoreType.DMA,
        pltpu.SemaphoreType.DMA,
        pltpu.SemaphoreType.DMA,
    ],
)
def _embed_sum(gsn_ids_hbm, route_ids_hbm, gsn_hbm, route_hbm, out_hbm,
               gidx, ridx, gbuf0, rbuf0, obuf0, gbuf1, rbuf1, obuf1,
               sem_g0, sem_g1, sem_r0, sem_r1, sem_o0, sem_o1):
    wid = lax.axis_index("s") * _NC + lax.axis_index("c")
    row_base = wid * _ROWS_W

    pltpu.sync_copy(gsn_ids_hbm.at[wid], gidx)
    pltpu.sync_copy(route_ids_hbm.at[wid], ridx)

    def add_rows(gbuf, rbuf, obuf):
        def row_body(i, c):
            for j in range(_HIDDEN // _LANES):
                sl = pl.ds(j * _LANES, _LANES)
                obuf[i, sl] = gbuf[i, sl] + rbuf[i, sl]
            return c
        lax.fori_loop(0, _L, row_body, 0)

    def start_gathers(b, gbuf, rbuf, sg, sr):
        isl = pl.ds(b * _LP, _LG)
        dg = pltpu.async_copy(gsn_hbm.at[gidx.at[isl]], gbuf, sg)
        dr = pltpu.async_copy(route_hbm.at[ridx.at[isl]], rbuf, sr)
        return dg, dr

    def start_writeback(b, obuf, so):
        return pltpu.async_copy(obuf, out_hbm.at[row_base + b], so)

    def group_body(g, carry):
        b0 = 4 * g
        b1 = b0 + 1
        b2 = b0 + 2
        b3 = b0 + 3
        dg0, dr0 = start_gathers(b0, gbuf0, rbuf0, sem_g0, sem_r0)
        dg1, dr1 = start_gathers(b1, gbuf1, rbuf1, sem_g1, sem_r1)

        dg0.wait()
        dr0.wait()
        add_rows(gbuf0, rbuf0, obuf0)
        wb0 = start_writeback(b0, obuf0, sem_o0)
        dg2, dr2 = start_gathers(b2, gbuf0, rbuf0, sem_g0, sem_r0)

        dg1.wait()
        dr1.wait()
        add_rows(gbuf1, rbuf1, obuf1)
        wb1 = start_writeback(b1, obuf1, sem_o1)
        dg3, dr3 = start_gathers(b3, gbuf1, rbuf1, sem_g1, sem_r1)

        wb0.wait()
        dg2.wait()
        dr2.wait()
        add_rows(gbuf0, rbuf0, obuf0)
        wb2 = start_writeback(b2, obuf0, sem_o0)

        wb1.wait()
        dg3.wait()
        dr3.wait()
        add_rows(gbuf1, rbuf1, obuf1)
        wb3 = start_writeback(b3, obuf1, sem_o1)

        wb2.wait()
        wb3.wait()
        return carry

    lax.fori_loop(0, _ROWS_W // 4, group_body, 0)


def _pad_ids(ids):
    ids = ids.astype(jnp.int32)
    ids = jnp.pad(ids, ((0, 0), (0, _LP - _L)), mode='edge')
    return ids.reshape(_NW, _ROWS_W * _LP)


def kernel(gsn_ids, route_ids, gsn_table, route_table):
    return _embed_sum(_pad_ids(gsn_ids), _pad_ids(route_ids),
                      gsn_table, route_table)


# speedup vs baseline: 2.3200x; 1.0698x over previous
import functools

import jax
import jax.numpy as jnp
from jax import lax
from jax.experimental import pallas as pl
from jax.experimental.pallas import tpu as pltpu
from jax.experimental.pallas import tpu_sc as plsc

_HIDDEN = 768
_B, _L = 4096, 20
_LP = 32
_LG = 24

_NC, _NS, _LANES = 2, 16, 16
_NW = _NC * _NS
_ROWS_W = _B // _NW

_mesh = plsc.VectorSubcoreMesh(core_axis_name="c", subcore_axis_name="s")


@functools.partial(
    pl.kernel,
    mesh=_mesh,
    out_type=jax.ShapeDtypeStruct((_B, _L, _HIDDEN), jnp.float32),
    scratch_types=[
        pltpu.VMEM((_ROWS_W * _LP,), jnp.int32),
        pltpu.VMEM((_ROWS_W * _LP,), jnp.int32),
        pltpu.VMEM((_LG, _HIDDEN), jnp.float32),
        pltpu.VMEM((_LG, _HIDDEN), jnp.float32),
        pltpu.VMEM((_L, _HIDDEN), jnp.float32),
        pltpu.VMEM((_LG, _HIDDEN), jnp.float32),
        pltpu.VMEM((_LG, _HIDDEN), jnp.float32),
        pltpu.VMEM((_L, _HIDDEN), jnp.float32),
        pltpu.SemaphoreType.DMA,
        pltpu.SemaphoreType.DMA,
        pltpu.SemaphoreType.DMA,
        pltpu.SemaphoreType.DMA,
        pltpu.SemaphoreType.DMA,
        pltpu.SemaphoreType.DMA,
    ],
)
def _embed_sum(gsn_ids_hbm, route_ids_hbm, gsn_hbm, route_hbm, out_hbm,
               gidx, ridx, gbuf0, rbuf0, obuf0, gbuf1, rbuf1, obuf1,
               sem_g0, sem_g1, sem_r0, sem_r1, sem_o0, sem_o1):
    wid = lax.axis_index("s") * _NC + lax.axis_index("c")
    row_base = wid * _ROWS_W

    pltpu.sync_copy(gsn_ids_hbm.at[wid], gidx)
    pltpu.sync_copy(route_ids_hbm.at[wid], ridx)

    def add_rows(gbuf, rbuf, obuf):
        def row_body(i, c):
            for j in range(_HIDDEN // _LANES):
                sl = pl.ds(j * _LANES, _LANES)
                obuf[i, sl] = gbuf[i, sl] + rbuf[i, sl]
            return c
        lax.fori_loop(0, _L, row_body, 0)

    def start_gathers(b, gbuf, rbuf, sg, sr):
        isl = pl.ds(b * _LP, _LG)
        dg = pltpu.async_copy(gsn_hbm.at[gidx.at[isl]], gbuf, sg)
        dr = pltpu.async_copy(route_hbm.at[ridx.at[isl]], rbuf, sr)
        return dg, dr

    def start_writeback(b, obuf, so):
        return pltpu.async_copy(obuf, out_hbm.at[row_base + b], so)

    _KPG = 8
    sets = ((gbuf0, rbuf0, obuf0, sem_g0, sem_r0, sem_o0),
            (gbuf1, rbuf1, obuf1, sem_g1, sem_r1, sem_o1))

    def group_body(g, carry):
        base_b = _KPG * g
        gds = {}
        wbs = {}
        for k in range(2):
            gb, rb, _, sg, sr, _ = sets[k]
            gds[k] = start_gathers(base_b + k, gb, rb, sg, sr)
        for k in range(_KPG):
            gb, rb, ob, sg, sr, so = sets[k % 2]
            if k >= 2:
                wbs[k - 2].wait()
            dg, dr = gds.pop(k)
            dg.wait()
            dr.wait()
            add_rows(gb, rb, ob)
            wbs[k] = start_writeback(base_b + k, ob, so)
            if k + 2 < _KPG:
                gds[k + 2] = start_gathers(base_b + k + 2, gb, rb, sg, sr)
        wbs[_KPG - 2].wait()
        wbs[_KPG - 1].wait()
        return carry

    lax.fori_loop(0, _ROWS_W // _KPG, group_body, 0)


def _pad_ids(ids):
    ids = ids.astype(jnp.int32)
    ids = jnp.pad(ids, ((0, 0), (0, _LP - _L)), mode='edge')
    return ids.reshape(_NW, _ROWS_W * _LP)


def kernel(gsn_ids, route_ids, gsn_table, route_table):
    return _embed_sum(_pad_ids(gsn_ids), _pad_ids(route_ids),
                      gsn_table, route_table)
